# Initial kernel scaffold; baseline (speedup 1.0000x reference)
import jax
import jax.numpy as jnp
from jax.experimental import pallas as pl


def kernel(x, embedding, fc_w, fc_b):
    raise NotImplementedError("write your pallas kernel here")


# speedup vs baseline: 1.7231x; 1.7231x over previous
import functools

import jax
import jax.numpy as jnp
from jax import lax
from jax.experimental import pallas as pl
from jax.experimental.pallas import tpu as pltpu
from jax.experimental.pallas import tpu_sc as plsc

VOCAB = 1000000
EMBED = 32
BATCH = 4096
SEQ = 200

NC = 2
NS = 16
NW = NC * NS
B_PER_W = BATCH // NW
CHUNK = 100
CHUNKS_PER_ROW = SEQ // CHUNK
N_CHUNKS = B_PER_W * CHUNKS_PER_ROW


@functools.partial(
    pl.kernel,
    mesh=plsc.VectorSubcoreMesh(core_axis_name="c", subcore_axis_name="s"),
    out_type=jax.ShapeDtypeStruct((BATCH,), jnp.float32),
    compiler_params=pltpu.CompilerParams(
        needs_layout_passes=False, use_tc_tiling_on_sc=False),
    scratch_types=[
        pltpu.VMEM((N_CHUNKS, CHUNK), jnp.int32),
        pltpu.VMEM((CHUNK, EMBED), jnp.float32),
        pltpu.VMEM((B_PER_W,), jnp.float32),
        pltpu.VMEM((48,), jnp.float32),
        pltpu.SemaphoreType.DMA,
    ],
)
def _sc_classify(x_hbm, params_hbm, emb_hbm, out_hbm,
                 idx_v, rows_v, out_v, par_v, sem):
    wid = lax.axis_index("s") * NC + lax.axis_index("c")
    base = wid * B_PER_W

    pltpu.sync_copy(x_hbm.at[wid], idx_v)
    pltpu.sync_copy(params_hbm, par_v)

    w0 = par_v[pl.ds(0, 16)]
    w1 = par_v[pl.ds(16, 16)]
    bias = par_v[pl.ds(32, 16)][0]
    zeros = jnp.zeros((16,), jnp.float32)

    lanes = lax.broadcasted_iota(jnp.int32, (16,), 0)

    def group_body(g, carry):
        def row_body(i, z_vec):
            r = g * 16 + i

            def half_body(h, accs):
                c = r * CHUNKS_PER_ROW + h
                pltpu.async_copy(emb_hbm.at[idx_v.at[c]], rows_v, sem).wait()

                def red_body(j, accs2):
                    a0, a1 = accs2
                    a0 = a0 + rows_v[j, pl.ds(0, 16)]
                    a1 = a1 + rows_v[j, pl.ds(16, 16)]
                    return (a0, a1)

                return lax.fori_loop(0, CHUNK, red_body, accs)

            acc0, acc1 = lax.fori_loop(0, CHUNKS_PER_ROW, half_body,
                                       (zeros, zeros))
            z = jnp.sum(acc0 * w0) + jnp.sum(acc1 * w1)
            z = z * (1.0 / SEQ) + bias
            return jnp.where(lanes == i, z, z_vec)

        z_vec = lax.fori_loop(0, 16, row_body, zeros)
        out_v[pl.ds(g * 16, 16)] = 1.0 / (1.0 + jnp.exp(-z_vec))
        return carry

    lax.fori_loop(0, B_PER_W // 16, group_body, 0)

    pltpu.sync_copy(out_v, out_hbm.at[pl.ds(base, B_PER_W)])


def kernel(x, embedding, fc_w, fc_b):
    x_r = x.astype(jnp.int32).reshape(NW, N_CHUNKS, CHUNK)
    params = jnp.concatenate(
        [fc_w.reshape(-1), fc_b.reshape(-1),
         jnp.zeros((15,), jnp.float32)]).astype(jnp.float32)
    out = _sc_classify(x_r, params, embedding)
    return out.reshape(BATCH, 1)


# speedup vs baseline: 2.1621x; 1.2548x over previous
import functools

import jax
import jax.numpy as jnp
from jax import lax
from jax.experimental import pallas as pl
from jax.experimental.pallas import tpu as pltpu
from jax.experimental.pallas import tpu_sc as plsc

VOCAB = 1000000
EMBED = 32
BATCH = 4096
SEQ = 200

NC = 2
NS = 16
NW = NC * NS
B_PER_W = BATCH // NW
CHUNK = 100
CHUNKS_PER_ROW = SEQ // CHUNK
N_CHUNKS = B_PER_W * CHUNKS_PER_ROW


@functools.partial(
    pl.kernel,
    mesh=plsc.VectorSubcoreMesh(core_axis_name="c", subcore_axis_name="s"),
    out_type=jax.ShapeDtypeStruct((BATCH,), jnp.float32),
    compiler_params=pltpu.CompilerParams(
        needs_layout_passes=False, use_tc_tiling_on_sc=False),
    scratch_types=[
        pltpu.VMEM((N_CHUNKS, CHUNK), jnp.int32),
        pltpu.VMEM((CHUNK, EMBED), jnp.float32),
        pltpu.VMEM((CHUNK, EMBED), jnp.float32),
        pltpu.VMEM((B_PER_W,), jnp.float32),
        pltpu.VMEM((48,), jnp.float32),
        pltpu.SemaphoreType.DMA,
        pltpu.SemaphoreType.DMA,
    ],
)
def _sc_classify(x_hbm, params_hbm, emb_hbm, out_hbm,
                 idx_v, buf0_v, buf1_v, out_v, par_v, sem0, sem1):
    wid = lax.axis_index("s") * NC + lax.axis_index("c")
    base = wid * B_PER_W

    pltpu.sync_copy(x_hbm.at[wid], idx_v)
    pltpu.sync_copy(params_hbm, par_v)

    w0 = par_v[pl.ds(0, 16)]
    w1 = par_v[pl.ds(16, 16)]
    bias = par_v[pl.ds(32, 16)][0]
    zeros = jnp.zeros((16,), jnp.float32)
    lanes = lax.broadcasted_iota(jnp.int32, (16,), 0)

    def gather(c, buf, sem):
        return pltpu.make_async_copy(emb_hbm.at[idx_v.at[c]], buf, sem)

    def reduce_chunk(buf, accs):
        def red_body(j, accs2):
            a0, a1 = accs2
            a0 = a0 + buf[j, pl.ds(0, 16)]
            a1 = a1 + buf[j, pl.ds(16, 16)]
            return (a0, a1)
        return lax.fori_loop(0, CHUNK, red_body, accs, unroll=10)

    gather(0, buf0_v, sem0).start()

    def row_body(r, z_vec):
        c = r * 2
        gather(c + 1, buf1_v, sem1).start()
        gather(c, buf0_v, sem0).wait()
        accs = reduce_chunk(buf0_v, (zeros, zeros))
        @pl.when(r < B_PER_W - 1)
        def _():
            gather(c + 2, buf0_v, sem0).start()
        gather(c + 1, buf1_v, sem1).wait()
        acc0, acc1 = reduce_chunk(buf1_v, accs)

        z = jnp.sum(acc0 * w0) + jnp.sum(acc1 * w1)
        z = z * (1.0 / SEQ) + bias
        z_vec = jnp.where(lanes == (r % 16), z, z_vec)

        @pl.when(r % 16 == 15)
        def _():
            out_v[pl.ds((r // 16) * 16, 16)] = 1.0 / (1.0 + jnp.exp(-z_vec))

        return z_vec

    lax.fori_loop(0, B_PER_W, row_body, zeros)

    pltpu.sync_copy(out_v, out_hbm.at[pl.ds(base, B_PER_W)])


def kernel(x, embedding, fc_w, fc_b):
    x_r = x.astype(jnp.int32).reshape(NW, N_CHUNKS, CHUNK)
    params = jnp.concatenate(
        [fc_w.reshape(-1), fc_b.reshape(-1),
         jnp.zeros((15,), jnp.float32)]).astype(jnp.float32)
    out = _sc_classify(x_r, params, embedding)
    return out.reshape(BATCH, 1)


# speedup vs baseline: 2.3096x; 1.0682x over previous
import functools

import jax
import jax.numpy as jnp
from jax import lax
from jax.experimental import pallas as pl
from jax.experimental.pallas import tpu as pltpu
from jax.experimental.pallas import tpu_sc as plsc

VOCAB = 1000000
EMBED = 32
BATCH = 4096
SEQ = 200

NC = 2
NS = 16
NW = NC * NS
B_PER_W = BATCH // NW
CA = 104
CB = SEQ - CA


@functools.partial(
    pl.kernel,
    mesh=plsc.VectorSubcoreMesh(core_axis_name="c", subcore_axis_name="s"),
    out_type=jax.ShapeDtypeStruct((BATCH,), jnp.float32),
    compiler_params=pltpu.CompilerParams(
        needs_layout_passes=False, use_tc_tiling_on_sc=False),
    scratch_types=[
        pltpu.VMEM((B_PER_W, SEQ), jnp.int32),
        pltpu.VMEM((CA, EMBED), jnp.float32),
        pltpu.VMEM((CA, EMBED), jnp.float32),
        pltpu.VMEM((CB, EMBED), jnp.float32),
        pltpu.VMEM((CB, EMBED), jnp.float32),
        pltpu.VMEM((B_PER_W,), jnp.float32),
        pltpu.VMEM((48,), jnp.float32),
        pltpu.SemaphoreType.DMA,
        pltpu.SemaphoreType.DMA,
        pltpu.SemaphoreType.DMA,
        pltpu.SemaphoreType.DMA,
    ],
)
def _sc_classify(x_hbm, params_hbm, emb_hbm, out_hbm,
                 idx_v, a0_v, a1_v, b0_v, b1_v, out_v, par_v,
                 sa0, sa1, sb0, sb1):
    wid = lax.axis_index("s") * NC + lax.axis_index("c")
    base = wid * B_PER_W

    pltpu.sync_copy(x_hbm.at[pl.ds(base, B_PER_W)], idx_v)
    pltpu.sync_copy(params_hbm, par_v)

    w0 = par_v[pl.ds(0, 16)]
    w1 = par_v[pl.ds(16, 16)]
    bias = par_v[pl.ds(32, 16)][0]
    zeros = jnp.zeros((16,), jnp.float32)
    lanes = lax.broadcasted_iota(jnp.int32, (16,), 0)

    def gather_a(r, buf, sem):
        return pltpu.make_async_copy(
            emb_hbm.at[idx_v.at[r, pl.ds(0, CA)]], buf, sem)

    def gather_b(r, buf, sem):
        return pltpu.make_async_copy(
            emb_hbm.at[idx_v.at[r, pl.ds(CA, CB)]], buf, sem)

    def reduce_chunk(buf, n, accs):
        def red_body(j, accs2):
            a0, a1 = accs2
            a0 = a0 + buf[j, pl.ds(0, 16)]
            a1 = a1 + buf[j, pl.ds(16, 16)]
            return (a0, a1)
        return lax.fori_loop(0, n, red_body, accs, unroll=8)

    def finalize(r, acc0, acc1, z_vec):
        z = jnp.sum(acc0 * w0) + jnp.sum(acc1 * w1)
        z = z * (1.0 / SEQ) + bias
        z_vec = jnp.where(lanes == (r % 16), z, z_vec)

        @pl.when(r % 16 == 15)
        def _():
            out_v[pl.ds((r // 16) * 16, 16)] = 1.0 / (1.0 + jnp.exp(-z_vec))

        return z_vec

    gather_a(0, a0_v, sa0).start()
    gather_b(0, b0_v, sb0).start()

    def pair_body(k, z_vec):
        r0 = k * 2
        r1 = r0 + 1
        gather_a(r1, a1_v, sa1).start()
        gather_a(r0, a0_v, sa0).wait()
        accs = reduce_chunk(a0_v, CA, (zeros, zeros))
        gather_b(r1, b1_v, sb1).start()
        gather_b(r0, b0_v, sb0).wait()
        acc0, acc1 = reduce_chunk(b0_v, CB, accs)
        z_vec = finalize(r0, acc0, acc1, z_vec)

        @pl.when(r1 < B_PER_W - 1)
        def _():
            gather_a(r1 + 1, a0_v, sa0).start()
        gather_a(r1, a1_v, sa1).wait()
        accs = reduce_chunk(a1_v, CA, (zeros, zeros))

        @pl.when(r1 < B_PER_W - 1)
        def _():
            gather_b(r1 + 1, b0_v, sb0).start()
        gather_b(r1, b1_v, sb1).wait()
        acc0, acc1 = reduce_chunk(b1_v, CB, accs)
        return finalize(r1, acc0, acc1, z_vec)

    lax.fori_loop(0, B_PER_W // 2, pair_body, zeros)

    pltpu.sync_copy(out_v, out_hbm.at[pl.ds(base, B_PER_W)])


def kernel(x, embedding, fc_w, fc_b):
    params = jnp.concatenate(
        [fc_w.reshape(-1), fc_b.reshape(-1),
         jnp.zeros((15,), jnp.float32)]).astype(jnp.float32)
    out = _sc_classify(x.astype(jnp.int32), params, embedding)
    return out.reshape(BATCH, 1)
